# baseline (speedup 1.0000x reference)
import functools

import jax
import jax.numpy as jnp
from jax import lax
from jax.experimental import pallas as pl
from jax.experimental.pallas import tpu as pltpu
from jax.experimental.pallas import tpu_sc as plsc

NC = 2
NS = 16
NW = NC * NS
D = 64
C = 128
KT = 2048


def _prep_table(et, bias_f):
    v = et.shape[1]
    grid = (v + KT - 1) // KT
    eye = jnp.eye(D, dtype=jnp.float32)

    def body(a_ref, b_ref, e_ref, o_ref):
        o_ref[...] = lax.dot_general(
            a_ref[...], e_ref[...],
            (((0,), (0,)), ((), ())),
            preferred_element_type=jnp.float32,
        ) + b_ref[...].reshape(KT, 1)

    return pl.pallas_call(
        body,
        grid=(grid,),
        in_specs=[
            pl.BlockSpec((D, KT), lambda j: (0, j)),
            pl.BlockSpec((KT,), lambda j: (j,)),
            pl.BlockSpec((D, D), lambda j: (0, 0)),
        ],
        out_specs=pl.BlockSpec((KT, D), lambda j: (j, 0)),
        out_shape=jax.ShapeDtypeStruct((v, D), jnp.float32),
    )(et, bias_f, eye)


def _gather(n_w, xf, tbl):
    n = xf.shape[0]
    n_chunks = n_w // C
    mesh = plsc.VectorSubcoreMesh(
        core_axis_name="c", subcore_axis_name="s", num_cores=NC, num_subcores=NS
    )

    @functools.partial(
        pl.kernel,
        out_type=jax.ShapeDtypeStruct((n, D), jnp.float32),
        mesh=mesh,
        scratch_types=[
            pltpu.VMEM((C,), jnp.int32),
            pltpu.VMEM((C,), jnp.int32),
            pltpu.VMEM((C, D), jnp.float32),
            pltpu.VMEM((C, D), jnp.float32),
            pltpu.SemaphoreType.DMA,
            pltpu.SemaphoreType.DMA,
            pltpu.SemaphoreType.DMA,
            pltpu.SemaphoreType.DMA,
        ],
        compiler_params=pltpu.CompilerParams(use_tc_tiling_on_sc=False),
    )
    def run(xf_hbm, tbl_hbm, out_hbm, idx0, idx1, row0, row1, gs0, gs1, os0, os1):
        idx_v = (idx0, idx1)
        rows_v = (row0, row1)
        gsem = (gs0, gs1)
        osem = (os0, os1)

        wid = lax.axis_index("s") * NC + lax.axis_index("c")
        base = wid * n_w

        def gather_copy(par):
            return pltpu.make_async_copy(
                tbl_hbm.at[idx_v[par]], rows_v[par], gsem[par])

        def fire(c, par):
            m0 = base + c * C
            pltpu.sync_copy(xf_hbm.at[pl.ds(m0, C)], idx_v[par])
            gather_copy(par).start()

        def out_copy(c, par):
            m0 = base + c * C
            return pltpu.make_async_copy(
                rows_v[par], out_hbm.at[pl.ds(m0, C)], osem[par])

        fire(0, 0)

        def chunk(c, par):
            @pl.when(c >= 1)
            def _():
                out_copy(c - 1, 1 - par).wait()

            @pl.when(c < n_chunks - 1)
            def _():
                fire(c + 1, 1 - par)

            gather_copy(par).wait()
            out_copy(c, par).start()

        def pair_body(h, _):
            chunk(2 * h, 0)
            chunk(2 * h + 1, 1)
            return 0

        lax.fori_loop(0, n_chunks // 2, pair_body, 0, unroll=False)
        out_copy(n_chunks - 1, 1).wait()

    return run(xf, tbl)


def kernel(x, embedding_weight, bias_weight):
    b, f = x.shape
    n = b * f
    d = embedding_weight.shape[1]
    assert n % NW == 0 and (n // NW) % (2 * C) == 0 and d == D
    xf = x.reshape(n)
    et = embedding_weight.T
    bias_flat = bias_weight.reshape(-1)
    tbl = _prep_table(et, bias_flat)
    out = _gather(n // NW, xf, tbl)
    return out.reshape(b, f, D)


# speedup vs baseline: 1.3459x; 1.3459x over previous
import functools

import jax
import jax.numpy as jnp
from jax import lax
from jax.experimental import pallas as pl
from jax.experimental.pallas import tpu as pltpu
from jax.experimental.pallas import tpu_sc as plsc

NC = 2
NS = 16
NW = NC * NS
D = 64
C = 128
L = 16


def _run(n_w, xf, emb, bias_f, interpret=False):
    n = xf.shape[0]
    n_chunks = n_w // C
    mesh = plsc.VectorSubcoreMesh(
        core_axis_name="c", subcore_axis_name="s", num_cores=NC, num_subcores=NS
    )

    @functools.partial(
        pl.kernel,
        out_type=jax.ShapeDtypeStruct((n, D), jnp.float32),
        mesh=mesh,
        scratch_types=[
            pltpu.VMEM((C,), jnp.int32),
            pltpu.VMEM((C,), jnp.int32),
            pltpu.VMEM((C,), jnp.float32),
            pltpu.VMEM((C,), jnp.float32),
            pltpu.VMEM((C, D), jnp.float32),
            pltpu.VMEM((C, D), jnp.float32),
            pltpu.VMEM((C, D), jnp.float32),
            pltpu.VMEM((C, D), jnp.float32),
            pltpu.SemaphoreType.DMA,
            pltpu.SemaphoreType.DMA,
            pltpu.SemaphoreType.DMA,
            pltpu.SemaphoreType.DMA,
        ],
        interpret=interpret,
        compiler_params=pltpu.CompilerParams(use_tc_tiling_on_sc=False),
    )
    def run(xf_hbm, emb_hbm, bias_hbm, out_hbm,
            idx0, idx1, bia0, bia1, row0, row1, obu0, obu1,
            gs0, gs1, os0, os1):
        idx_v = (idx0, idx1)
        bias_v = (bia0, bia1)
        rows_v = (row0, row1)
        obuf_v = (obu0, obu1)
        gsem = (gs0, gs1)
        osem = (os0, os1)

        wid = lax.axis_index("s") * NC + lax.axis_index("c")
        base = wid * n_w

        def gather_copies(par):
            return [
                pltpu.make_async_copy(
                    emb_hbm.at[idx_v[par]], rows_v[par], gsem[par]),
                pltpu.make_async_copy(
                    bias_hbm.at[idx_v[par]], bias_v[par], gsem[par]),
            ]

        def fire(c, par):
            m0 = base + c * C
            pltpu.sync_copy(xf_hbm.at[pl.ds(m0, C)], idx_v[par])
            for cp in gather_copies(par):
                cp.start()

        def out_copy(c, par):
            m0 = base + c * C
            return pltpu.make_async_copy(
                obuf_v[par], out_hbm.at[pl.ds(m0, C)], osem[par])

        def compute(par):
            def row_body(r16, _):
                bv16 = bias_v[par][pl.ds(r16 * L, L)]
                for j in range(L):
                    r = r16 * L + j
                    bv = jnp.broadcast_to(bv16[j], (L,))
                    for q in range(D // L):
                        obuf_v[par][r, pl.ds(q * L, L)] = (
                            rows_v[par][r, pl.ds(q * L, L)] + bv)
                return 0

            lax.fori_loop(0, C // L, row_body, 0, unroll=False)

        fire(0, 0)

        def chunk(c, par):
            @pl.when(c < n_chunks - 1)
            def _():
                fire(c + 1, 1 - par)

            for cp in gather_copies(par):
                cp.wait()

            @pl.when(c >= 2)
            def _():
                out_copy(c - 2, par).wait()

            compute(par)
            out_copy(c, par).start()

        def pair_body(h, _):
            chunk(2 * h, 0)
            chunk(2 * h + 1, 1)
            return 0

        lax.fori_loop(0, n_chunks // 2, pair_body, 0, unroll=False)
        out_copy(n_chunks - 2, 0).wait()
        out_copy(n_chunks - 1, 1).wait()

    return run(xf, emb, bias_f)


def kernel(x, embedding_weight, bias_weight):
    b, f = x.shape
    n = b * f
    d = embedding_weight.shape[1]
    assert n % NW == 0 and (n // NW) % (2 * C) == 0 and d == D
    xf = x.reshape(n)
    bias_flat = bias_weight.reshape(-1)
    out = _run(n // NW, xf, embedding_weight, bias_flat)
    return out.reshape(b, f, D)
